# baseline (speedup 1.0000x reference)
import functools

import jax
import jax.numpy as jnp
from jax import lax
from jax.experimental import pallas as pl
from jax.experimental.pallas import tpu as pltpu
from jax.experimental.pallas import tpu_sc as plsc

NUM_CORES = 2
NUM_SUBCORES = 16
LANES = 16
BATCH = 16384
HALF = BATCH // NUM_CORES
HIDDEN = 64
KREG = HIDDEN // LANES
NUM_NODES = 1000000
RANGE = 62464
WSZ = 512
TAIL_START = 999936
TAIL_NL = TAIL_START - 15 * RANGE
B_PER_W = 512
PSTRIDE = B_PER_W + 1
SENTINEL = (65535 << 14) | 8191


def _sc_body(node_t, tail_t, rel_t, hidx, ridx, tidx, out, rows, stage, sem):
    c = lax.axis_index("c")
    s = lax.axis_index("s")
    s15 = s == NUM_SUBCORES - 1
    lo = s * RANGE
    hi = jnp.where(s15, NUM_NODES, lo + RANGE)
    iota = lax.iota(jnp.int32, LANES)

    def process_win(win, k, cnt, wl):
        nvreg = (cnt + LANES - 1) // LANES

        @pl.loop(0, nvreg)
        def _vreg(v):
            e16 = wl[pl.ds(v * LANES, LANES)]
            inw = jnp.where((e16 >> 14) >> 9 == k, 1, 0)
            tot = plsc.all_reduce_population_count(inw == 1)

            @pl.when(tot[0] > 0)
            def _():
                for li in range(LANES):
                    @pl.when(inw[li] == 1)
                    def _():
                        e = e16[li]
                        nloc = (e >> 14) - k * WSZ
                        nv = jnp.full((LANES,), nloc, jnp.int32)
                        for k4 in range(KREG):
                            g = plsc.load_gather(win, [iota + k4 * LANES, nv])
                            stage[pl.ds(li * HIDDEN + k4 * LANES, LANES)] = g
                        dst = ((e >> 13) & 1) * BATCH + c * HALF + (e & 8191)
                        pltpu.async_copy(
                            stage.at[pl.ds(li * HIDDEN, HIDDEN)],
                            rows.at[pl.ds(dst * HIDDEN, HIDDEN)], sem)
                for li in range(LANES):
                    @pl.when(inw[li] == 1)
                    def _():
                        e = e16[li]
                        dst = ((e >> 13) & 1) * BATCH + c * HALF + (e & 8191)
                        pltpu.make_async_copy(
                            stage.at[pl.ds(li * HIDDEN, HIDDEN)],
                            rows.at[pl.ds(dst * HIDDEN, HIDDEN)], sem).wait()

    def scan_phase(idx_all, wl, win, tailbuf):
        pltpu.sync_copy(hidx.at[pl.ds(c * HALF, HALF)], idx_all.at[pl.ds(0, HALF)])
        pltpu.sync_copy(tidx.at[pl.ds(c * HALF, HALF)], idx_all.at[pl.ds(HALF, HALF)])
        pltpu.sync_copy(tail_t, tailbuf)

        @pl.loop(0, BATCH // LANES, init_carry=0)
        def _match(i, cnt):
            n = idx_all[pl.ds(i * LANES, LANES)]
            m = (n >= lo) & (n < hi)
            mi = jnp.where(m, 1, 0)
            excl = plsc.cumsum(mi) - mi
            half_i = i % (HALF // LANES)
            flag = i // (HALF // LANES)
            entry = ((n - lo) << 14) | (flag << 13) | (half_i * LANES + iota)
            plsc.store_scatter(wl, [cnt + excl], entry, mask=m)
            tot = plsc.all_reduce_population_count(m)
            return cnt + tot[0]

        cnt = _match
        plsc.store_scatter(wl, [cnt + iota],
                           jnp.full((LANES,), SENTINEL, jnp.int32))

        nwin = jnp.where(s15, RANGE // WSZ + 1, RANGE // WSZ)

        @pl.loop(0, nwin)
        def _win(k):
            pltpu.sync_copy(node_t.at[:, pl.ds(lo + k * WSZ, WSZ)], win)
            process_win(win, k, cnt, wl)

        @pl.when(s15)
        def _():
            process_win(tailbuf, TAIL_NL // WSZ, cnt, wl)

    pl.run_scoped(
        scan_phase,
        pltpu.VMEM((BATCH + LANES,), jnp.int32),
        pltpu.VMEM((BATCH + LANES,), jnp.int32),
        pltpu.VMEM((HIDDEN, WSZ), jnp.float32),
        pltpu.VMEM((HIDDEN, HIDDEN), jnp.float32),
    )

    plsc.subcore_barrier()

    base = c * HALF + s * B_PER_W

    def rel_phase(relbuf, ridx_loc, rloc):
        pltpu.sync_copy(rel_t, relbuf)
        pltpu.sync_copy(ridx.at[pl.ds(base, B_PER_W)],
                        ridx_loc.at[pl.ds(0, B_PER_W)])

        @pl.loop(0, B_PER_W)
        def _item(i):
            r = ridx_loc[pl.ds(i, LANES)][0]
            nv = jnp.full((LANES,), r, jnp.int32)
            for k4 in range(KREG):
                g = plsc.load_gather(relbuf, [iota + k4 * LANES, nv])
                rloc[pl.ds(i * HIDDEN + k4 * LANES, LANES)] = g

    def score_phase(rloc, hloc, tloc, pt, out_v):
        pltpu.sync_copy(rows.at[pl.ds((c * HALF + s * B_PER_W) * HIDDEN,
                                      B_PER_W * HIDDEN)], hloc)
        pltpu.sync_copy(rows.at[pl.ds((BATCH + c * HALF + s * B_PER_W) * HIDDEN,
                                      B_PER_W * HIDDEN)], tloc)

        @pl.loop(0, B_PER_W)
        def _row(i):
            acc = None
            for k4 in range(KREG):
                d = pl.ds(i * HIDDEN + k4 * LANES, LANES)
                v = jnp.abs(hloc[d] + rloc[d] - tloc[d])
                acc = v if acc is None else acc + v
            plsc.store_scatter(pt, [iota * PSTRIDE + i], acc)

        @pl.loop(0, B_PER_W // LANES)
        def _grp(g):
            gb = g * LANES
            sv = pt[pl.ds(gb, LANES)]
            for l in range(1, LANES):
                sv = sv + pt[pl.ds(l * PSTRIDE + gb, LANES)]
            out_v[pl.ds(gb, LANES)] = -sv

        pltpu.sync_copy(out_v, out.at[pl.ds(base, B_PER_W)])

    def late_phases(rloc):
        pl.run_scoped(
            functools.partial(rel_phase, rloc=rloc),
            pltpu.VMEM((HIDDEN, 1000), jnp.float32),
            pltpu.VMEM((B_PER_W + LANES,), jnp.int32),
        )
        pl.run_scoped(
            functools.partial(score_phase, rloc),
            pltpu.VMEM((B_PER_W * HIDDEN,), jnp.float32),
            pltpu.VMEM((B_PER_W * HIDDEN,), jnp.float32),
            pltpu.VMEM((LANES * PSTRIDE,), jnp.float32),
            pltpu.VMEM((B_PER_W,), jnp.float32),
        )

    pl.run_scoped(late_phases,
                  pltpu.VMEM((B_PER_W * HIDDEN,), jnp.float32))


_mesh = plsc.VectorSubcoreMesh(
    core_axis_name="c", subcore_axis_name="s",
    num_cores=NUM_CORES, num_subcores=NUM_SUBCORES)

_sc_call = functools.partial(
    pl.kernel,
    out_type=(
        jax.ShapeDtypeStruct((BATCH,), jnp.float32),
        jax.ShapeDtypeStruct((2 * BATCH * HIDDEN,), jnp.float32),
    ),
    mesh=_mesh,
    compiler_params=pltpu.CompilerParams(
        needs_layout_passes=False, use_tc_tiling_on_sc=True),
    scratch_types=[
        pltpu.VMEM((LANES * HIDDEN,), jnp.float32),
        pltpu.SemaphoreType.DMA,
    ],
)(_sc_body)


@jax.jit
def kernel(head_index, rel_type, tail_index, node_emb, rel_emb):
    h = head_index.astype(jnp.int32)
    r = rel_type.astype(jnp.int32)
    t = tail_index.astype(jnp.int32)
    node_t = node_emb.T
    tail_t = node_emb.T[:, TAIL_START:]
    rel_t = rel_emb.T
    score, _ = _sc_call(node_t, tail_t, rel_t, h, r, t)
    return score


# speedup vs baseline: 1.1540x; 1.1540x over previous
import functools

import jax
import jax.numpy as jnp
from jax import lax
from jax.experimental import pallas as pl
from jax.experimental.pallas import tpu as pltpu
from jax.experimental.pallas import tpu_sc as plsc

NUM_CORES = 2
NUM_SUBCORES = 16
LANES = 16
BATCH = 16384
HALF = BATCH // NUM_CORES
HIDDEN = 64
KREG = HIDDEN // LANES
NUM_NODES = 1000000
RANGE = 62464
WSZ = 512
TAIL_START = 999936
TAIL_NL = TAIL_START - 15 * RANGE
B_PER_W = 512
PSTRIDE = B_PER_W + 1
SENTINEL = (65535 << 14) | 8191


def _sc_body(node_t, tail_t, rel_t, hidx, ridx, tidx, out, rows, stage, sem,
             sem2):
    c = lax.axis_index("c")
    s = lax.axis_index("s")
    s15 = s == NUM_SUBCORES - 1
    lo = s * RANGE
    hi = jnp.where(s15, NUM_NODES, lo + RANGE)
    iota = lax.iota(jnp.int32, LANES)
    NSLOT = 64

    def process_win(win, k, cnt, wl, fd):
        nvreg = (cnt + LANES - 1) // LANES

        @pl.loop(0, nvreg, init_carry=fd)
        def _vreg(v, fd):
            fires, drained = fd
            e16 = wl[pl.ds(v * LANES, LANES)]
            inw = jnp.where((e16 >> 14) >> 9 == k, 1, 0)
            ranks = plsc.cumsum(inw) - inw
            tot = plsc.all_reduce_population_count(inw == 1)[0]

            @pl.when(tot > 0)
            def _():
                for li in range(LANES):
                    @pl.when(inw[li] == 1)
                    def _():
                        e = e16[li]
                        nloc = (e >> 14) - k * WSZ
                        slot = lax.rem(fires + ranks[li], NSLOT)
                        nv = jnp.full((LANES,), nloc, jnp.int32)
                        for k4 in range(KREG):
                            g = plsc.load_gather(win, [iota + k4 * LANES, nv])
                            stage[pl.ds(slot * HIDDEN + k4 * LANES, LANES)] = g
                        dst = ((e >> 13) & 1) * BATCH + c * HALF + (e & 8191)
                        pltpu.async_copy(
                            stage.at[pl.ds(slot * HIDDEN, HIDDEN)],
                            rows.at[pl.ds(dst * HIDDEN, HIDDEN)], sem)

            fires = fires + tot
            ndrain = jnp.maximum(fires - drained - (NSLOT - LANES), 0)

            @pl.loop(0, ndrain)
            def _d(_):
                pltpu.make_async_copy(
                    stage.at[pl.ds(0, HIDDEN)],
                    rows.at[pl.ds(0, HIDDEN)], sem).wait()

            return (fires, drained + ndrain)

        return _vreg

    def scan_phase(idx_all, wl, win_a, win_b, tailbuf):
        pltpu.sync_copy(hidx.at[pl.ds(c * HALF, HALF)], idx_all.at[pl.ds(0, HALF)])
        pltpu.sync_copy(tidx.at[pl.ds(c * HALF, HALF)], idx_all.at[pl.ds(HALF, HALF)])
        pltpu.sync_copy(tail_t, tailbuf)

        @pl.loop(0, BATCH // LANES, init_carry=0)
        def _match(i, cnt):
            n = idx_all[pl.ds(i * LANES, LANES)]
            m = (n >= lo) & (n < hi)
            mi = jnp.where(m, 1, 0)
            excl = plsc.cumsum(mi) - mi
            half_i = i % (HALF // LANES)
            flag = i // (HALF // LANES)
            entry = ((n - lo) << 14) | (flag << 13) | (half_i * LANES + iota)
            plsc.store_scatter(wl, [cnt + excl], entry, mask=m)
            tot = plsc.all_reduce_population_count(m)
            return cnt + tot[0]

        cnt = _match
        plsc.store_scatter(wl, [cnt + iota],
                           jnp.full((LANES,), SENTINEL, jnp.int32))

        def start(k, buf):
            pltpu.async_copy(node_t.at[:, pl.ds(lo + k * WSZ, WSZ)], buf, sem2)

        def wait_win(buf):
            pltpu.make_async_copy(
                node_t.at[:, pl.ds(0, WSZ)], buf, sem2).wait()

        start(0, win_a)

        @pl.loop(0, RANGE // WSZ // 2, init_carry=(jnp.int32(0), jnp.int32(0)))
        def _pair(k2, fd):
            k = 2 * k2
            wait_win(win_a)
            start(k + 1, win_b)
            fd = process_win(win_a, k, cnt, wl, fd)
            wait_win(win_b)
            start(k + 2, win_a)
            fd = process_win(win_b, k + 1, cnt, wl, fd)
            return fd

        wait_win(win_a)
        fd = process_win(win_a, RANGE // WSZ, cnt, wl, _pair)
        fd = process_win(tailbuf, TAIL_NL // WSZ, cnt, wl, fd)

        fires, drained = fd

        @pl.loop(0, fires - drained)
        def _dfin(_):
            pltpu.make_async_copy(
                stage.at[pl.ds(0, HIDDEN)],
                rows.at[pl.ds(0, HIDDEN)], sem).wait()

    pl.run_scoped(
        scan_phase,
        pltpu.VMEM((BATCH + LANES,), jnp.int32),
        pltpu.VMEM((BATCH + LANES,), jnp.int32),
        pltpu.VMEM((HIDDEN, WSZ), jnp.float32),
        pltpu.VMEM((HIDDEN, WSZ), jnp.float32),
        pltpu.VMEM((HIDDEN, HIDDEN), jnp.float32),
    )

    plsc.subcore_barrier()

    base = c * HALF + s * B_PER_W

    def rel_phase(relbuf, ridx_loc, rloc):
        pltpu.sync_copy(rel_t, relbuf)
        pltpu.sync_copy(ridx.at[pl.ds(base, B_PER_W)],
                        ridx_loc.at[pl.ds(0, B_PER_W)])

        @pl.loop(0, B_PER_W)
        def _item(i):
            r = ridx_loc[pl.ds(i, LANES)][0]
            nv = jnp.full((LANES,), r, jnp.int32)
            for k4 in range(KREG):
                g = plsc.load_gather(relbuf, [iota + k4 * LANES, nv])
                rloc[pl.ds(i * HIDDEN + k4 * LANES, LANES)] = g

    def score_phase(rloc, hloc, tloc, pt, out_v):
        pltpu.sync_copy(rows.at[pl.ds((c * HALF + s * B_PER_W) * HIDDEN,
                                      B_PER_W * HIDDEN)], hloc)
        pltpu.sync_copy(rows.at[pl.ds((BATCH + c * HALF + s * B_PER_W) * HIDDEN,
                                      B_PER_W * HIDDEN)], tloc)

        @pl.loop(0, B_PER_W)
        def _row(i):
            acc = None
            for k4 in range(KREG):
                d = pl.ds(i * HIDDEN + k4 * LANES, LANES)
                v = jnp.abs(hloc[d] + rloc[d] - tloc[d])
                acc = v if acc is None else acc + v
            plsc.store_scatter(pt, [iota * PSTRIDE + i], acc)

        @pl.loop(0, B_PER_W // LANES)
        def _grp(g):
            gb = g * LANES
            sv = pt[pl.ds(gb, LANES)]
            for l in range(1, LANES):
                sv = sv + pt[pl.ds(l * PSTRIDE + gb, LANES)]
            out_v[pl.ds(gb, LANES)] = -sv

        pltpu.sync_copy(out_v, out.at[pl.ds(base, B_PER_W)])

    def late_phases(rloc):
        pl.run_scoped(
            functools.partial(rel_phase, rloc=rloc),
            pltpu.VMEM((HIDDEN, 1000), jnp.float32),
            pltpu.VMEM((B_PER_W + LANES,), jnp.int32),
        )
        pl.run_scoped(
            functools.partial(score_phase, rloc),
            pltpu.VMEM((B_PER_W * HIDDEN,), jnp.float32),
            pltpu.VMEM((B_PER_W * HIDDEN,), jnp.float32),
            pltpu.VMEM((LANES * PSTRIDE,), jnp.float32),
            pltpu.VMEM((B_PER_W,), jnp.float32),
        )

    pl.run_scoped(late_phases,
                  pltpu.VMEM((B_PER_W * HIDDEN,), jnp.float32))


_mesh = plsc.VectorSubcoreMesh(
    core_axis_name="c", subcore_axis_name="s",
    num_cores=NUM_CORES, num_subcores=NUM_SUBCORES)

_sc_call = functools.partial(
    pl.kernel,
    out_type=(
        jax.ShapeDtypeStruct((BATCH,), jnp.float32),
        jax.ShapeDtypeStruct((2 * BATCH * HIDDEN,), jnp.float32),
    ),
    mesh=_mesh,
    compiler_params=pltpu.CompilerParams(
        needs_layout_passes=False, use_tc_tiling_on_sc=True),
    scratch_types=[
        pltpu.VMEM((64 * HIDDEN,), jnp.float32),
        pltpu.SemaphoreType.DMA,
        pltpu.SemaphoreType.DMA,
    ],
)(_sc_body)


@jax.jit
def kernel(head_index, rel_type, tail_index, node_emb, rel_emb):
    h = head_index.astype(jnp.int32)
    r = rel_type.astype(jnp.int32)
    t = tail_index.astype(jnp.int32)
    node_t = node_emb.T
    tail_t = node_emb.T[:, TAIL_START:]
    rel_t = rel_emb.T
    score, _ = _sc_call(node_t, tail_t, rel_t, h, r, t)
    return score


# speedup vs baseline: 3.2300x; 2.7990x over previous
import functools

import jax
import jax.numpy as jnp
from jax import lax
from jax.experimental import pallas as pl
from jax.experimental.pallas import tpu as pltpu
from jax.experimental.pallas import tpu_sc as plsc

NUM_CORES = 2
NUM_SUBCORES = 16
LANES = 16
BATCH = 16384
HALF = BATCH // NUM_CORES
HIDDEN = 64
KREG = HIDDEN // LANES
NUM_NODES = 1000000
RANGE = 62464
WSZ = 512
TAIL_START = 999936
TAIL_NL = TAIL_START - 15 * RANGE
B_PER_W = 512
PSTRIDE = B_PER_W + 1
SENTINEL = (65535 << 14) | 8191
HCNT = 128


def _sc_body(node_t, tail_t, rel_t, hidx, ridx, tidx, out, rows, stage, sem,
             sem2):
    c = lax.axis_index("c")
    s = lax.axis_index("s")
    s15 = s == NUM_SUBCORES - 1
    lo = s * RANGE
    hi = jnp.where(s15, NUM_NODES, lo + RANGE)
    iota = lax.iota(jnp.int32, LANES)
    NSLOT = 64

    def process_win(win, k, base, wl2, fd):
        b0 = base[pl.ds(k, LANES)][0]
        b1 = base[pl.ds(k + 1, LANES)][0]

        @pl.loop(b0, b1, init_carry=fd)
        def _ent(e, fd):
            fires, drained = fd
            ev = wl2[pl.ds(e, LANES)][0]
            nloc = (ev >> 14) - k * WSZ
            slot = lax.rem(fires, NSLOT)
            nv = jnp.full((LANES,), nloc, jnp.int32)
            for k4 in range(KREG):
                g = plsc.load_gather(win, [iota + k4 * LANES, nv])
                stage[pl.ds(slot * HIDDEN + k4 * LANES, LANES)] = g
            dst = ((ev >> 13) & 1) * BATCH + c * HALF + (ev & 8191)
            pltpu.async_copy(
                stage.at[pl.ds(slot * HIDDEN, HIDDEN)],
                rows.at[pl.ds(dst * HIDDEN, HIDDEN)], sem)
            fires = fires + 1
            ndrain = jnp.maximum(fires - drained - (NSLOT - LANES), 0)

            @pl.loop(0, ndrain)
            def _d(_):
                pltpu.make_async_copy(
                    stage.at[pl.ds(0, HIDDEN)],
                    rows.at[pl.ds(0, HIDDEN)], sem).wait()

            return (fires, drained + ndrain)

        return _ent

    def scan_phase(idx_all, wl, win_a, win_b, tailbuf, hist, base, off, tmp):
        pltpu.sync_copy(hidx.at[pl.ds(c * HALF, HALF)], idx_all.at[pl.ds(0, HALF)])
        pltpu.sync_copy(tidx.at[pl.ds(c * HALF, HALF)], idx_all.at[pl.ds(HALF, HALF)])
        pltpu.sync_copy(tail_t, tailbuf)

        @pl.loop(0, BATCH // LANES, init_carry=0)
        def _match(i, cnt):
            n = idx_all[pl.ds(i * LANES, LANES)]
            m = (n >= lo) & (n < hi)
            mi = jnp.where(m, 1, 0)
            excl = plsc.cumsum(mi) - mi
            half_i = i % (HALF // LANES)
            flag = i // (HALF // LANES)
            entry = ((n - lo) << 14) | (flag << 13) | (half_i * LANES + iota)
            plsc.store_scatter(wl, [cnt + excl], entry, mask=m)
            tot = plsc.all_reduce_population_count(m)
            return cnt + tot[0]

        cnt = _match
        plsc.store_scatter(wl, [cnt + iota],
                           jnp.full((LANES,), SENTINEL, jnp.int32))
        nvreg = (cnt + LANES - 1) // LANES

        zeros = jnp.zeros((LANES,), jnp.int32)
        for h8 in range(HCNT // LANES):
            hist[pl.ds(h8 * LANES, LANES)] = zeros
        tmp[pl.ds(0, LANES)] = jnp.full((LANES,), -1, jnp.int32)
        tmp[pl.ds(LANES, LANES)] = jnp.full((LANES,), 1 << 30, jnp.int32)

        def sorted_ranks(v):
            e16 = wl[pl.ds(v * LANES, LANES)]
            swid, sent = plsc.sort_key_val(e16 >> 23, e16)
            tmp[pl.ds(1, LANES)] = swid
            prev = tmp[pl.ds(0, LANES)]
            nxt = tmp[pl.ds(2, LANES)]
            neq = jnp.where(swid != prev, 1, 0)
            is_last = swid != nxt
            seg0 = plsc.cummax(jnp.where(neq == 1, iota, 0))
            rank = iota - seg0
            return swid, sent, rank, is_last

        @pl.loop(0, nvreg)
        def _hist(v):
            swid, _, rank, is_last = sorted_ranks(v)
            plsc.addupdate_scatter(hist, [swid], rank + 1, mask=is_last)

        @pl.loop(0, HCNT // LANES, init_carry=0)
        def _pref(h8, run):
            hv = hist[pl.ds(h8 * LANES, LANES)]
            cs = plsc.cumsum(hv)
            ex = cs - hv + run
            base[pl.ds(h8 * LANES, LANES)] = ex
            off[pl.ds(h8 * LANES, LANES)] = ex
            return run + cs[LANES - 1]

        @pl.loop(0, nvreg)
        def _scat(v):
            swid, sent, rank, is_last = sorted_ranks(v)
            pos = plsc.load_gather(off, [swid]) + rank
            plsc.store_scatter(idx_all, [pos], sent)
            plsc.addupdate_scatter(off, [swid], rank + 1, mask=is_last)

        wl2 = idx_all

        def start(k, buf):
            pltpu.async_copy(node_t.at[:, pl.ds(lo + k * WSZ, WSZ)], buf, sem2)

        def wait_win(buf):
            pltpu.make_async_copy(
                node_t.at[:, pl.ds(0, WSZ)], buf, sem2).wait()

        start(0, win_a)

        @pl.loop(0, RANGE // WSZ // 2, init_carry=(jnp.int32(0), jnp.int32(0)))
        def _pair(k2, fd):
            k = 2 * k2
            wait_win(win_a)
            start(k + 1, win_b)
            fd = process_win(win_a, k, base, wl2, fd)
            wait_win(win_b)
            start(k + 2, win_a)
            fd = process_win(win_b, k + 1, base, wl2, fd)
            return fd

        wait_win(win_a)
        fd = process_win(win_a, RANGE // WSZ, base, wl2, _pair)
        fd = process_win(tailbuf, TAIL_NL // WSZ, base, wl2, fd)

        fires, drained = fd

        @pl.loop(0, fires - drained)
        def _dfin(_):
            pltpu.make_async_copy(
                stage.at[pl.ds(0, HIDDEN)],
                rows.at[pl.ds(0, HIDDEN)], sem).wait()

    pl.run_scoped(
        scan_phase,
        pltpu.VMEM((BATCH + LANES,), jnp.int32),
        pltpu.VMEM((BATCH + LANES,), jnp.int32),
        pltpu.VMEM((HIDDEN, WSZ), jnp.float32),
        pltpu.VMEM((HIDDEN, WSZ), jnp.float32),
        pltpu.VMEM((HIDDEN, HIDDEN), jnp.float32),
        pltpu.VMEM((HCNT + LANES,), jnp.int32),
        pltpu.VMEM((HCNT + LANES,), jnp.int32),
        pltpu.VMEM((HCNT + LANES,), jnp.int32),
        pltpu.VMEM((2 * LANES,), jnp.int32),
    )

    plsc.subcore_barrier()

    base = c * HALF + s * B_PER_W

    def rel_phase(relbuf, ridx_loc, rloc):
        pltpu.sync_copy(rel_t, relbuf)
        pltpu.sync_copy(ridx.at[pl.ds(base, B_PER_W)],
                        ridx_loc.at[pl.ds(0, B_PER_W)])

        @pl.loop(0, B_PER_W)
        def _item(i):
            r = ridx_loc[pl.ds(i, LANES)][0]
            nv = jnp.full((LANES,), r, jnp.int32)
            for k4 in range(KREG):
                g = plsc.load_gather(relbuf, [iota + k4 * LANES, nv])
                rloc[pl.ds(i * HIDDEN + k4 * LANES, LANES)] = g

    def score_phase(rloc, hloc, tloc, pt, out_v):
        pltpu.sync_copy(rows.at[pl.ds((c * HALF + s * B_PER_W) * HIDDEN,
                                      B_PER_W * HIDDEN)], hloc)
        pltpu.sync_copy(rows.at[pl.ds((BATCH + c * HALF + s * B_PER_W) * HIDDEN,
                                      B_PER_W * HIDDEN)], tloc)

        @pl.loop(0, B_PER_W)
        def _row(i):
            acc = None
            for k4 in range(KREG):
                d = pl.ds(i * HIDDEN + k4 * LANES, LANES)
                v = jnp.abs(hloc[d] + rloc[d] - tloc[d])
                acc = v if acc is None else acc + v
            plsc.store_scatter(pt, [iota * PSTRIDE + i], acc)

        @pl.loop(0, B_PER_W // LANES)
        def _grp(g):
            gb = g * LANES
            sv = pt[pl.ds(gb, LANES)]
            for l in range(1, LANES):
                sv = sv + pt[pl.ds(l * PSTRIDE + gb, LANES)]
            out_v[pl.ds(gb, LANES)] = -sv

        pltpu.sync_copy(out_v, out.at[pl.ds(base, B_PER_W)])

    def late_phases(rloc):
        pl.run_scoped(
            functools.partial(rel_phase, rloc=rloc),
            pltpu.VMEM((HIDDEN, 1000), jnp.float32),
            pltpu.VMEM((B_PER_W + LANES,), jnp.int32),
        )
        pl.run_scoped(
            functools.partial(score_phase, rloc),
            pltpu.VMEM((B_PER_W * HIDDEN,), jnp.float32),
            pltpu.VMEM((B_PER_W * HIDDEN,), jnp.float32),
            pltpu.VMEM((LANES * PSTRIDE,), jnp.float32),
            pltpu.VMEM((B_PER_W,), jnp.float32),
        )

    pl.run_scoped(late_phases,
                  pltpu.VMEM((B_PER_W * HIDDEN,), jnp.float32))


_mesh = plsc.VectorSubcoreMesh(
    core_axis_name="c", subcore_axis_name="s",
    num_cores=NUM_CORES, num_subcores=NUM_SUBCORES)

_sc_call = functools.partial(
    pl.kernel,
    out_type=(
        jax.ShapeDtypeStruct((BATCH,), jnp.float32),
        jax.ShapeDtypeStruct((2 * BATCH * HIDDEN,), jnp.float32),
    ),
    mesh=_mesh,
    compiler_params=pltpu.CompilerParams(
        needs_layout_passes=False, use_tc_tiling_on_sc=True),
    scratch_types=[
        pltpu.VMEM((64 * HIDDEN,), jnp.float32),
        pltpu.SemaphoreType.DMA,
        pltpu.SemaphoreType.DMA,
    ],
)(_sc_body)


@jax.jit
def kernel(head_index, rel_type, tail_index, node_emb, rel_emb):
    h = head_index.astype(jnp.int32)
    r = rel_type.astype(jnp.int32)
    t = tail_index.astype(jnp.int32)
    node_t = node_emb.T
    tail_t = node_emb.T[:, TAIL_START:]
    rel_t = rel_emb.T
    score, _ = _sc_call(node_t, tail_t, rel_t, h, r, t)
    return score


# speedup vs baseline: 4.7485x; 1.4701x over previous
import functools

import jax
import jax.numpy as jnp
from jax import lax
from jax.experimental import pallas as pl
from jax.experimental.pallas import tpu as pltpu
from jax.experimental.pallas import tpu_sc as plsc

NUM_CORES = 2
NUM_SUBCORES = 16
NW = NUM_CORES * NUM_SUBCORES
LANES = 16
BATCH = 16384
HIDDEN = 64
KREG = HIDDEN // LANES
NUM_NODES = 1000000
RANGE = 31232
WSZ = 256
NWIN = RANGE // WSZ
TAIL_START = 999936
TAIL_NL = TAIL_START - (NW - 1) * RANGE
TAIL_WID = TAIL_NL // WSZ
B_PER_W = BATCH // NW
PSTRIDE = B_PER_W + 1
SENTINEL = (65535 << 15) | 16383
HCNT = 256
NSLOT = 64


def _scan_body(node_t, tail_t, hidx, tidx, rows, stage, sem, sem2):
    c = lax.axis_index("c")
    s = lax.axis_index("s")
    w = s * NUM_CORES + c
    lo = w * RANGE
    hi = jnp.where(w == NW - 1, NUM_NODES, lo + RANGE)
    iota = lax.iota(jnp.int32, LANES)

    def process_win(win, k, base, wl2, fd):
        b0 = base[pl.ds(k, LANES)][0]
        b1 = base[pl.ds(k + 1, LANES)][0]

        @pl.loop(b0, b1, init_carry=fd)
        def _ent(e, fd):
            fires, drained = fd
            ev = wl2[pl.ds(e, LANES)][0]
            nloc = (ev >> 15) - k * WSZ
            slot = lax.rem(fires, NSLOT)
            nv = jnp.full((LANES,), nloc, jnp.int32)
            for k4 in range(KREG):
                g = plsc.load_gather(win, [iota + k4 * LANES, nv])
                stage[pl.ds(slot * HIDDEN + k4 * LANES, LANES)] = g
            dst = ((ev >> 14) & 1) * BATCH + (ev & 16383)
            pltpu.async_copy(
                stage.at[pl.ds(slot * HIDDEN, HIDDEN)],
                rows.at[pl.ds(dst * HIDDEN, HIDDEN)], sem)
            fires = fires + 1
            ndrain = jnp.maximum(fires - drained - (NSLOT - LANES), 0)

            @pl.loop(0, ndrain)
            def _d(_):
                pltpu.make_async_copy(
                    stage.at[pl.ds(0, HIDDEN)],
                    rows.at[pl.ds(0, HIDDEN)], sem).wait()

            return (fires, drained + ndrain)

        return _ent

    def scan_phase(idx_all, wl, win_a, win_b, tailbuf, hist, base, off, tmp):
        def start(k, buf):
            pltpu.async_copy(node_t.at[:, pl.ds(lo + k * WSZ, WSZ)], buf, sem2)

        def wait_win(buf):
            pltpu.make_async_copy(
                node_t.at[:, pl.ds(0, WSZ)], buf, sem2).wait()

        start(0, win_a)
        start(1, win_b)

        pltpu.sync_copy(hidx, idx_all.at[pl.ds(0, BATCH)])
        pltpu.sync_copy(tidx, idx_all.at[pl.ds(BATCH, BATCH)])
        pltpu.sync_copy(tail_t, tailbuf)

        @pl.loop(0, 2 * BATCH // LANES, init_carry=0)
        def _match(i, cnt):
            n = idx_all[pl.ds(i * LANES, LANES)]
            m = (n >= lo) & (n < hi)
            mi = jnp.where(m, 1, 0)
            excl = plsc.cumsum(mi) - mi
            item_i = i % (BATCH // LANES)
            flag = i // (BATCH // LANES)
            entry = ((n - lo) << 15) | (flag << 14) | (item_i * LANES + iota)
            plsc.store_scatter(wl, [cnt + excl], entry, mask=m)
            tot = plsc.all_reduce_population_count(m)
            return cnt + tot[0]

        cnt = _match
        plsc.store_scatter(wl, [cnt + iota],
                           jnp.full((LANES,), SENTINEL, jnp.int32))
        nvreg = (cnt + LANES - 1) // LANES

        zeros = jnp.zeros((LANES,), jnp.int32)
        for h8 in range(HCNT // LANES):
            hist[pl.ds(h8 * LANES, LANES)] = zeros
        tmp[pl.ds(0, LANES)] = jnp.full((LANES,), -1, jnp.int32)
        tmp[pl.ds(LANES, LANES)] = jnp.full((LANES,), 1 << 30, jnp.int32)

        def sorted_ranks(v):
            e16 = wl[pl.ds(v * LANES, LANES)]
            swid, sent = plsc.sort_key_val(e16 >> 23, e16)
            tmp[pl.ds(1, LANES)] = swid
            prev = tmp[pl.ds(0, LANES)]
            nxt = tmp[pl.ds(2, LANES)]
            neq = jnp.where(swid != prev, 1, 0)
            is_last = swid != nxt
            seg0 = plsc.cummax(jnp.where(neq == 1, iota, 0))
            rank = iota - seg0
            return swid, sent, rank, is_last

        @pl.loop(0, nvreg)
        def _hist(v):
            swid, _, rank, is_last = sorted_ranks(v)
            plsc.addupdate_scatter(hist, [swid], rank + 1, mask=is_last)

        @pl.loop(0, HCNT // LANES, init_carry=0)
        def _pref(h8, run):
            hv = hist[pl.ds(h8 * LANES, LANES)]
            cs = plsc.cumsum(hv)
            ex = cs - hv + run
            base[pl.ds(h8 * LANES, LANES)] = ex
            off[pl.ds(h8 * LANES, LANES)] = ex
            return run + cs[LANES - 1]

        @pl.loop(0, nvreg)
        def _scat(v):
            swid, sent, rank, is_last = sorted_ranks(v)
            pos = plsc.load_gather(off, [swid]) + rank
            plsc.store_scatter(idx_all, [pos], sent)
            plsc.addupdate_scatter(off, [swid], rank + 1, mask=is_last)

        wl2 = idx_all

        @pl.loop(0, NWIN // 2, init_carry=(jnp.int32(0), jnp.int32(0)))
        def _pair(k2, fd):
            k = 2 * k2
            wait_win(win_a)
            fd = process_win(win_a, k, base, wl2, fd)
            start(k + 2, win_a)
            wait_win(win_b)
            fd = process_win(win_b, k + 1, base, wl2, fd)
            start(k + 3, win_b)
            return fd

        wait_win(win_a)
        fd = process_win(win_a, NWIN, base, wl2, _pair)
        wait_win(win_b)
        fd = process_win(win_b, NWIN + 1, base, wl2, fd)
        fd = process_win(tailbuf, TAIL_WID, base, wl2, fd)

        fires, drained = fd

        @pl.loop(0, fires - drained)
        def _dfin(_):
            pltpu.make_async_copy(
                stage.at[pl.ds(0, HIDDEN)],
                rows.at[pl.ds(0, HIDDEN)], sem).wait()

    pl.run_scoped(
        scan_phase,
        pltpu.VMEM((2 * BATCH + LANES,), jnp.int32),
        pltpu.VMEM((2 * BATCH + LANES,), jnp.int32),
        pltpu.VMEM((HIDDEN, WSZ), jnp.float32),
        pltpu.VMEM((HIDDEN, WSZ), jnp.float32),
        pltpu.VMEM((HIDDEN, HIDDEN), jnp.float32),
        pltpu.VMEM((HCNT + LANES,), jnp.int32),
        pltpu.VMEM((HCNT + LANES,), jnp.int32),
        pltpu.VMEM((HCNT + LANES,), jnp.int32),
        pltpu.VMEM((2 * LANES,), jnp.int32),
    )


def _score_body(rows, rel_t, ridx, out, stage, sem, sem2):
    c = lax.axis_index("c")
    s = lax.axis_index("s")
    w = s * NUM_CORES + c
    base = w * B_PER_W
    iota = lax.iota(jnp.int32, LANES)

    def rel_phase(relbuf, ridx_loc, rloc):
        pltpu.sync_copy(rel_t, relbuf)
        pltpu.sync_copy(ridx.at[pl.ds(base, B_PER_W)],
                        ridx_loc.at[pl.ds(0, B_PER_W)])

        @pl.loop(0, B_PER_W)
        def _item(i):
            r = ridx_loc[pl.ds(i, LANES)][0]
            nv = jnp.full((LANES,), r, jnp.int32)
            for k4 in range(KREG):
                g = plsc.load_gather(relbuf, [iota + k4 * LANES, nv])
                rloc[pl.ds(i * HIDDEN + k4 * LANES, LANES)] = g

    def score_phase(rloc, hloc, tloc, pt, out_v):
        pltpu.sync_copy(rows.at[pl.ds(base * HIDDEN, B_PER_W * HIDDEN)], hloc)
        pltpu.sync_copy(rows.at[pl.ds((BATCH + base) * HIDDEN,
                                      B_PER_W * HIDDEN)], tloc)

        @pl.loop(0, B_PER_W)
        def _row(i):
            acc = None
            for k4 in range(KREG):
                d = pl.ds(i * HIDDEN + k4 * LANES, LANES)
                v = jnp.abs(hloc[d] + rloc[d] - tloc[d])
                acc = v if acc is None else acc + v
            plsc.store_scatter(pt, [iota * PSTRIDE + i], acc)

        @pl.loop(0, B_PER_W // LANES)
        def _grp(g):
            gb = g * LANES
            sv = pt[pl.ds(gb, LANES)]
            for l in range(1, LANES):
                sv = sv + pt[pl.ds(l * PSTRIDE + gb, LANES)]
            out_v[pl.ds(gb, LANES)] = -sv

        pltpu.sync_copy(out_v, out.at[pl.ds(base, B_PER_W)])

    def late_phases(rloc):
        pl.run_scoped(
            functools.partial(rel_phase, rloc=rloc),
            pltpu.VMEM((HIDDEN, 1000), jnp.float32),
            pltpu.VMEM((B_PER_W + LANES,), jnp.int32),
        )
        pl.run_scoped(
            functools.partial(score_phase, rloc),
            pltpu.VMEM((B_PER_W * HIDDEN,), jnp.float32),
            pltpu.VMEM((B_PER_W * HIDDEN,), jnp.float32),
            pltpu.VMEM((LANES * PSTRIDE,), jnp.float32),
            pltpu.VMEM((B_PER_W,), jnp.float32),
        )

    pl.run_scoped(late_phases,
                  pltpu.VMEM((B_PER_W * HIDDEN,), jnp.float32))


_mesh = plsc.VectorSubcoreMesh(
    core_axis_name="c", subcore_axis_name="s",
    num_cores=NUM_CORES, num_subcores=NUM_SUBCORES)

_params = pltpu.CompilerParams(
    needs_layout_passes=False, use_tc_tiling_on_sc=True)

_scan_call = functools.partial(
    pl.kernel,
    out_type=jax.ShapeDtypeStruct((2 * BATCH * HIDDEN,), jnp.float32),
    mesh=_mesh,
    compiler_params=_params,
    scratch_types=[
        pltpu.VMEM((NSLOT * HIDDEN,), jnp.float32),
        pltpu.SemaphoreType.DMA,
        pltpu.SemaphoreType.DMA,
    ],
)(_scan_body)

_score_call = functools.partial(
    pl.kernel,
    out_type=jax.ShapeDtypeStruct((BATCH,), jnp.float32),
    mesh=_mesh,
    compiler_params=_params,
    scratch_types=[
        pltpu.VMEM((NSLOT * HIDDEN,), jnp.float32),
        pltpu.SemaphoreType.DMA,
        pltpu.SemaphoreType.DMA,
    ],
)(_score_body)


@jax.jit
def kernel(head_index, rel_type, tail_index, node_emb, rel_emb):
    h = head_index.astype(jnp.int32)
    r = rel_type.astype(jnp.int32)
    t = tail_index.astype(jnp.int32)
    node_t = node_emb.T
    tail_t = node_emb.T[:, TAIL_START:]
    rel_t = rel_emb.T
    rows = _scan_call(node_t, tail_t, h, t)
    return _score_call(rows, rel_t, r)


# speedup vs baseline: 4.8429x; 1.0199x over previous
import functools

import jax
import jax.numpy as jnp
from jax import lax
from jax.experimental import pallas as pl
from jax.experimental.pallas import tpu as pltpu
from jax.experimental.pallas import tpu_sc as plsc

NUM_CORES = 2
NUM_SUBCORES = 16
NW = NUM_CORES * NUM_SUBCORES
LANES = 16
BATCH = 16384
HIDDEN = 64
KREG = HIDDEN // LANES
NUM_NODES = 1000000
RANGE = 31232
WSZ = 256
NWIN = RANGE // WSZ
TAIL_START = 999936
TAIL_NL = TAIL_START - (NW - 1) * RANGE
TAIL_WID = TAIL_NL // WSZ
B_PER_W = BATCH // NW
PSTRIDE = B_PER_W + 1
SENTINEL = (65535 << 15) | 16383
HCNT = 256
NSLOT = 64


def _scan_body(node_t, tail_t, hidx, tidx, rows, stage, sem, sem2):
    c = lax.axis_index("c")
    s = lax.axis_index("s")
    w = s * NUM_CORES + c
    lo = w * RANGE
    hi = jnp.where(w == NW - 1, NUM_NODES, lo + RANGE)
    iota = lax.iota(jnp.int32, LANES)

    def process_win(win, k, base, wl2, fd):
        b0 = base[pl.ds(k, LANES)][0]
        b1 = base[pl.ds(k + 1, LANES)][0]

        @pl.loop(b0, b1, init_carry=fd)
        def _ent(e, fd):
            fires, drained = fd
            ev = wl2[pl.ds(e, LANES)][0]
            nloc = (ev >> 15) - k * WSZ
            slot = lax.rem(fires, NSLOT)
            nv = jnp.full((LANES,), nloc, jnp.int32)
            for k4 in range(KREG):
                g = plsc.load_gather(win, [iota + k4 * LANES, nv])
                stage[pl.ds(slot * HIDDEN + k4 * LANES, LANES)] = g
            dst = ((ev >> 14) & 1) * BATCH + (ev & 16383)
            pltpu.async_copy(
                stage.at[pl.ds(slot * HIDDEN, HIDDEN)],
                rows.at[pl.ds(dst * HIDDEN, HIDDEN)], sem)
            fires = fires + 1
            ndrain = jnp.maximum(fires - drained - (NSLOT - LANES), 0)

            @pl.loop(0, ndrain)
            def _d(_):
                pltpu.make_async_copy(
                    stage.at[pl.ds(0, HIDDEN)],
                    rows.at[pl.ds(0, HIDDEN)], sem).wait()

            return (fires, drained + ndrain)

        return _ent

    def scan_phase(idx_all, wl, win_a, win_b, tailbuf, hist, base, off, tmp):
        def start(k, buf):
            pltpu.async_copy(node_t.at[:, pl.ds(lo + k * WSZ, WSZ)], buf, sem2)

        def wait_win(buf):
            pltpu.make_async_copy(
                node_t.at[:, pl.ds(0, WSZ)], buf, sem2).wait()

        start(0, win_a)
        start(1, win_b)

        pltpu.sync_copy(hidx, idx_all.at[pl.ds(0, BATCH)])
        pltpu.sync_copy(tidx, idx_all.at[pl.ds(BATCH, BATCH)])
        pltpu.sync_copy(tail_t, tailbuf)

        @pl.loop(0, 2 * BATCH // LANES, init_carry=0, unroll=4)
        def _match(i, cnt):
            n = idx_all[pl.ds(i * LANES, LANES)]
            m = (n >= lo) & (n < hi)
            item_i = i % (BATCH // LANES)
            flag = i // (BATCH // LANES)
            entry = ((n - lo) << 15) | (flag << 14) | (item_i * LANES + iota)
            plsc.store_compressed(wl.at[pl.ds(cnt, LANES)], entry, mask=m)
            tot = plsc.all_reduce_population_count(m)
            return cnt + tot[0]

        cnt = _match
        plsc.store_scatter(wl, [cnt + iota],
                           jnp.full((LANES,), SENTINEL, jnp.int32))
        nvreg = (cnt + LANES - 1) // LANES

        zeros = jnp.zeros((LANES,), jnp.int32)
        for h8 in range(HCNT // LANES):
            hist[pl.ds(h8 * LANES, LANES)] = zeros
        tmp[pl.ds(0, LANES)] = jnp.full((LANES,), -1, jnp.int32)
        tmp[pl.ds(LANES, LANES)] = jnp.full((LANES,), 1 << 30, jnp.int32)

        def sorted_ranks(v):
            e16 = wl[pl.ds(v * LANES, LANES)]
            swid, sent = plsc.sort_key_val(e16 >> 23, e16)
            tmp[pl.ds(1, LANES)] = swid
            prev = tmp[pl.ds(0, LANES)]
            nxt = tmp[pl.ds(2, LANES)]
            neq = jnp.where(swid != prev, 1, 0)
            is_last = swid != nxt
            seg0 = plsc.cummax(jnp.where(neq == 1, iota, 0))
            rank = iota - seg0
            return swid, sent, rank, is_last

        @pl.loop(0, nvreg)
        def _hist(v):
            swid, _, rank, is_last = sorted_ranks(v)
            plsc.addupdate_scatter(hist, [swid], rank + 1, mask=is_last)

        @pl.loop(0, HCNT // LANES, init_carry=0)
        def _pref(h8, run):
            hv = hist[pl.ds(h8 * LANES, LANES)]
            cs = plsc.cumsum(hv)
            ex = cs - hv + run
            base[pl.ds(h8 * LANES, LANES)] = ex
            off[pl.ds(h8 * LANES, LANES)] = ex
            return run + cs[LANES - 1]

        @pl.loop(0, nvreg)
        def _scat(v):
            swid, sent, rank, is_last = sorted_ranks(v)
            pos = plsc.load_gather(off, [swid]) + rank
            plsc.store_scatter(idx_all, [pos], sent)
            plsc.addupdate_scatter(off, [swid], rank + 1, mask=is_last)

        wl2 = idx_all

        @pl.loop(0, NWIN // 2, init_carry=(jnp.int32(0), jnp.int32(0)))
        def _pair(k2, fd):
            k = 2 * k2
            wait_win(win_a)
            fd = process_win(win_a, k, base, wl2, fd)
            start(k + 2, win_a)
            wait_win(win_b)
            fd = process_win(win_b, k + 1, base, wl2, fd)
            start(k + 3, win_b)
            return fd

        wait_win(win_a)
        fd = process_win(win_a, NWIN, base, wl2, _pair)
        wait_win(win_b)
        fd = process_win(win_b, NWIN + 1, base, wl2, fd)
        fd = process_win(tailbuf, TAIL_WID, base, wl2, fd)

        fires, drained = fd

        @pl.loop(0, fires - drained)
        def _dfin(_):
            pltpu.make_async_copy(
                stage.at[pl.ds(0, HIDDEN)],
                rows.at[pl.ds(0, HIDDEN)], sem).wait()

    pl.run_scoped(
        scan_phase,
        pltpu.VMEM((2 * BATCH + LANES,), jnp.int32),
        pltpu.VMEM((2 * BATCH + LANES,), jnp.int32),
        pltpu.VMEM((HIDDEN, WSZ), jnp.float32),
        pltpu.VMEM((HIDDEN, WSZ), jnp.float32),
        pltpu.VMEM((HIDDEN, HIDDEN), jnp.float32),
        pltpu.VMEM((HCNT + LANES,), jnp.int32),
        pltpu.VMEM((HCNT + LANES,), jnp.int32),
        pltpu.VMEM((HCNT + LANES,), jnp.int32),
        pltpu.VMEM((2 * LANES,), jnp.int32),
    )


def _score_body(rows, rel_t, ridx, out, stage, sem, sem2):
    c = lax.axis_index("c")
    s = lax.axis_index("s")
    w = s * NUM_CORES + c
    base = w * B_PER_W
    iota = lax.iota(jnp.int32, LANES)

    def rel_phase(relbuf, ridx_loc, rloc):
        pltpu.sync_copy(rel_t, relbuf)
        pltpu.sync_copy(ridx.at[pl.ds(base, B_PER_W)],
                        ridx_loc.at[pl.ds(0, B_PER_W)])

        @pl.loop(0, B_PER_W)
        def _item(i):
            r = ridx_loc[pl.ds(i, LANES)][0]
            nv = jnp.full((LANES,), r, jnp.int32)
            for k4 in range(KREG):
                g = plsc.load_gather(relbuf, [iota + k4 * LANES, nv])
                rloc[pl.ds(i * HIDDEN + k4 * LANES, LANES)] = g

    def score_phase(rloc, hloc, tloc, pt, out_v):
        pltpu.sync_copy(rows.at[pl.ds(base * HIDDEN, B_PER_W * HIDDEN)], hloc)
        pltpu.sync_copy(rows.at[pl.ds((BATCH + base) * HIDDEN,
                                      B_PER_W * HIDDEN)], tloc)

        @pl.loop(0, B_PER_W)
        def _row(i):
            acc = None
            for k4 in range(KREG):
                d = pl.ds(i * HIDDEN + k4 * LANES, LANES)
                v = jnp.abs(hloc[d] + rloc[d] - tloc[d])
                acc = v if acc is None else acc + v
            plsc.store_scatter(pt, [iota * PSTRIDE + i], acc)

        @pl.loop(0, B_PER_W // LANES)
        def _grp(g):
            gb = g * LANES
            sv = pt[pl.ds(gb, LANES)]
            for l in range(1, LANES):
                sv = sv + pt[pl.ds(l * PSTRIDE + gb, LANES)]
            out_v[pl.ds(gb, LANES)] = -sv

        pltpu.sync_copy(out_v, out.at[pl.ds(base, B_PER_W)])

    def late_phases(rloc):
        pl.run_scoped(
            functools.partial(rel_phase, rloc=rloc),
            pltpu.VMEM((HIDDEN, 1000), jnp.float32),
            pltpu.VMEM((B_PER_W + LANES,), jnp.int32),
        )
        pl.run_scoped(
            functools.partial(score_phase, rloc),
            pltpu.VMEM((B_PER_W * HIDDEN,), jnp.float32),
            pltpu.VMEM((B_PER_W * HIDDEN,), jnp.float32),
            pltpu.VMEM((LANES * PSTRIDE,), jnp.float32),
            pltpu.VMEM((B_PER_W,), jnp.float32),
        )

    pl.run_scoped(late_phases,
                  pltpu.VMEM((B_PER_W * HIDDEN,), jnp.float32))


_mesh = plsc.VectorSubcoreMesh(
    core_axis_name="c", subcore_axis_name="s",
    num_cores=NUM_CORES, num_subcores=NUM_SUBCORES)

_params = pltpu.CompilerParams(
    needs_layout_passes=False, use_tc_tiling_on_sc=True)

_scan_call = functools.partial(
    pl.kernel,
    out_type=jax.ShapeDtypeStruct((2 * BATCH * HIDDEN,), jnp.float32),
    mesh=_mesh,
    compiler_params=_params,
    scratch_types=[
        pltpu.VMEM((NSLOT * HIDDEN,), jnp.float32),
        pltpu.SemaphoreType.DMA,
        pltpu.SemaphoreType.DMA,
    ],
)(_scan_body)

_score_call = functools.partial(
    pl.kernel,
    out_type=jax.ShapeDtypeStruct((BATCH,), jnp.float32),
    mesh=_mesh,
    compiler_params=_params,
    scratch_types=[
        pltpu.VMEM((NSLOT * HIDDEN,), jnp.float32),
        pltpu.SemaphoreType.DMA,
        pltpu.SemaphoreType.DMA,
    ],
)(_score_body)


@jax.jit
def kernel(head_index, rel_type, tail_index, node_emb, rel_emb):
    h = head_index.astype(jnp.int32)
    r = rel_type.astype(jnp.int32)
    t = tail_index.astype(jnp.int32)
    node_t = node_emb.T
    tail_t = node_emb.T[:, TAIL_START:]
    rel_t = rel_emb.T
    rows = _scan_call(node_t, tail_t, h, t)
    return _score_call(rows, rel_t, r)


# speedup vs baseline: 4.8702x; 1.0056x over previous
import functools

import jax
import jax.numpy as jnp
from jax import lax
from jax.experimental import pallas as pl
from jax.experimental.pallas import tpu as pltpu
from jax.experimental.pallas import tpu_sc as plsc

NUM_CORES = 2
NUM_SUBCORES = 16
NW = NUM_CORES * NUM_SUBCORES
LANES = 16
BATCH = 16384
HIDDEN = 64
KREG = HIDDEN // LANES
NUM_NODES = 1000000
RANGE = 31232
WSZ = 256
NWIN = RANGE // WSZ
TAIL_START = 999936
TAIL_NL = TAIL_START - (NW - 1) * RANGE
TAIL_WID = TAIL_NL // WSZ
B_PER_W = BATCH // NW
PSTRIDE = B_PER_W + 1
SENTINEL = (65535 << 15) | 16383
HCNT = 256
NSLOT = 64


def _scan_body(node_t, tail_t, hidx, tidx, rows, stage, sem, sem2):
    c = lax.axis_index("c")
    s = lax.axis_index("s")
    w = s * NUM_CORES + c
    lo = w * RANGE
    hi = jnp.where(w == NW - 1, NUM_NODES, lo + RANGE)
    iota = lax.iota(jnp.int32, LANES)

    def process_win(win, k, base, wl2, fd):
        b0 = base[pl.ds(k, LANES)][0]
        b1 = base[pl.ds(k + 1, LANES)][0]

        @pl.loop(b0, b1, init_carry=fd)
        def _ent(e, fd):
            fires, drained = fd
            ev = wl2[pl.ds(e, LANES)][0]
            nloc = (ev >> 15) - k * WSZ
            slot = lax.rem(fires, NSLOT)
            nv = jnp.full((LANES,), nloc, jnp.int32)
            for k4 in range(KREG):
                g = plsc.load_gather(win, [iota + k4 * LANES, nv])
                stage[pl.ds(slot * HIDDEN + k4 * LANES, LANES)] = g
            dst = ((ev >> 14) & 1) * BATCH + (ev & 16383)
            pltpu.async_copy(
                stage.at[pl.ds(slot * HIDDEN, HIDDEN)],
                rows.at[pl.ds(dst * HIDDEN, HIDDEN)], sem)
            fires = fires + 1
            ndrain = jnp.maximum(fires - drained - (NSLOT - LANES), 0)

            @pl.loop(0, ndrain)
            def _d(_):
                pltpu.make_async_copy(
                    stage.at[pl.ds(0, HIDDEN)],
                    rows.at[pl.ds(0, HIDDEN)], sem).wait()

            return (fires, drained + ndrain)

        return _ent

    def scan_phase(idx_all, wl, win_a, win_b, tailbuf, hist, base, off, tmp):
        def start(k, buf):
            pltpu.async_copy(node_t.at[:, pl.ds(lo + k * WSZ, WSZ)], buf, sem2)

        def wait_win(buf):
            pltpu.make_async_copy(
                node_t.at[:, pl.ds(0, WSZ)], buf, sem2).wait()

        start(0, win_a)
        start(1, win_b)

        pltpu.sync_copy(hidx, idx_all.at[pl.ds(0, BATCH)])
        pltpu.sync_copy(tidx, idx_all.at[pl.ds(BATCH, BATCH)])
        pltpu.sync_copy(tail_t, tailbuf)

        @pl.loop(0, 2 * BATCH // LANES, init_carry=0, unroll=4)
        def _match(i, cnt):
            n = idx_all[pl.ds(i * LANES, LANES)]
            m = (n >= lo) & (n < hi)
            item_i = i % (BATCH // LANES)
            flag = i // (BATCH // LANES)
            entry = ((n - lo) << 15) | (flag << 14) | (item_i * LANES + iota)
            plsc.store_compressed(wl.at[pl.ds(cnt, LANES)], entry, mask=m)
            tot = plsc.all_reduce_population_count(m)
            return cnt + tot[0]

        cnt = _match
        plsc.store_scatter(wl, [cnt + iota],
                           jnp.full((LANES,), SENTINEL, jnp.int32))
        nvreg = (cnt + LANES - 1) // LANES

        zeros = jnp.zeros((LANES,), jnp.int32)
        for h8 in range(HCNT // LANES):
            hist[pl.ds(h8 * LANES, LANES)] = zeros
        tmp[pl.ds(0, LANES)] = jnp.full((LANES,), -1, jnp.int32)
        tmp[pl.ds(LANES, LANES)] = jnp.full((LANES,), 1 << 30, jnp.int32)

        def sorted_ranks(v):
            e16 = wl[pl.ds(v * LANES, LANES)]
            swid, sent = plsc.sort_key_val(e16 >> 23, e16)
            tmp[pl.ds(1, LANES)] = swid
            prev = tmp[pl.ds(0, LANES)]
            nxt = tmp[pl.ds(2, LANES)]
            neq = jnp.where(swid != prev, 1, 0)
            is_last = swid != nxt
            seg0 = plsc.cummax(jnp.where(neq == 1, iota, 0))
            rank = iota - seg0
            return swid, sent, rank, is_last

        @pl.loop(0, nvreg)
        def _hist(v):
            swid, _, rank, is_last = sorted_ranks(v)
            plsc.addupdate_scatter(hist, [swid], rank + 1, mask=is_last)

        @pl.loop(0, HCNT // LANES, init_carry=0)
        def _pref(h8, run):
            hv = hist[pl.ds(h8 * LANES, LANES)]
            cs = plsc.cumsum(hv)
            ex = cs - hv + run
            base[pl.ds(h8 * LANES, LANES)] = ex
            off[pl.ds(h8 * LANES, LANES)] = ex
            return run + cs[LANES - 1]

        @pl.loop(0, nvreg)
        def _scat(v):
            swid, sent, rank, is_last = sorted_ranks(v)
            pos = plsc.load_gather(off, [swid]) + rank
            plsc.store_scatter(idx_all, [pos], sent)
            plsc.addupdate_scatter(off, [swid], rank + 1, mask=is_last)

        wl2 = idx_all

        @pl.loop(0, NWIN // 2, init_carry=(jnp.int32(0), jnp.int32(0)))
        def _pair(k2, fd):
            k = 2 * k2
            wait_win(win_a)
            fd = process_win(win_a, k, base, wl2, fd)
            start(k + 2, win_a)
            wait_win(win_b)
            fd = process_win(win_b, k + 1, base, wl2, fd)
            start(k + 3, win_b)
            return fd

        wait_win(win_a)
        fd = process_win(win_a, NWIN, base, wl2, _pair)
        wait_win(win_b)
        fd = process_win(win_b, NWIN + 1, base, wl2, fd)
        fd = process_win(tailbuf, TAIL_WID, base, wl2, fd)

        fires, drained = fd

        @pl.loop(0, fires - drained)
        def _dfin(_):
            pltpu.make_async_copy(
                stage.at[pl.ds(0, HIDDEN)],
                rows.at[pl.ds(0, HIDDEN)], sem).wait()

    pl.run_scoped(
        scan_phase,
        pltpu.VMEM((2 * BATCH + LANES,), jnp.int32),
        pltpu.VMEM((2 * BATCH + LANES,), jnp.int32),
        pltpu.VMEM((HIDDEN, WSZ), jnp.float32),
        pltpu.VMEM((HIDDEN, WSZ), jnp.float32),
        pltpu.VMEM((HIDDEN, HIDDEN), jnp.float32),
        pltpu.VMEM((HCNT + LANES,), jnp.int32),
        pltpu.VMEM((HCNT + LANES,), jnp.int32),
        pltpu.VMEM((HCNT + LANES,), jnp.int32),
        pltpu.VMEM((2 * LANES,), jnp.int32),
    )


def _score_body(rows, rel_t, ridx, out, stage, sem, sem2):
    c = lax.axis_index("c")
    s = lax.axis_index("s")
    w = s * NUM_CORES + c
    base = w * B_PER_W
    iota = lax.iota(jnp.int32, LANES)

    def rel_phase(relbuf, ridx_loc, rloc):
        pltpu.sync_copy(rel_t, relbuf)
        pltpu.sync_copy(ridx.at[pl.ds(base, B_PER_W)],
                        ridx_loc.at[pl.ds(0, B_PER_W)])

        @pl.loop(0, B_PER_W, unroll=2)
        def _item(i):
            r = ridx_loc[pl.ds(i, LANES)][0]
            nv = jnp.full((LANES,), r, jnp.int32)
            for k4 in range(KREG):
                g = plsc.load_gather(relbuf, [iota + k4 * LANES, nv])
                rloc[pl.ds(i * HIDDEN + k4 * LANES, LANES)] = g

    def score_phase(rloc, hloc, tloc, pt, out_v):
        pltpu.sync_copy(rows.at[pl.ds(base * HIDDEN, B_PER_W * HIDDEN)], hloc)
        pltpu.sync_copy(rows.at[pl.ds((BATCH + base) * HIDDEN,
                                      B_PER_W * HIDDEN)], tloc)

        @pl.loop(0, B_PER_W, unroll=2)
        def _row(i):
            acc = None
            for k4 in range(KREG):
                d = pl.ds(i * HIDDEN + k4 * LANES, LANES)
                v = jnp.abs(hloc[d] + rloc[d] - tloc[d])
                acc = v if acc is None else acc + v
            plsc.store_scatter(pt, [iota * PSTRIDE + i], acc)

        @pl.loop(0, B_PER_W // LANES)
        def _grp(g):
            gb = g * LANES
            sv = pt[pl.ds(gb, LANES)]
            for l in range(1, LANES):
                sv = sv + pt[pl.ds(l * PSTRIDE + gb, LANES)]
            out_v[pl.ds(gb, LANES)] = -sv

        pltpu.sync_copy(out_v, out.at[pl.ds(base, B_PER_W)])

    def late_phases(rloc):
        pl.run_scoped(
            functools.partial(rel_phase, rloc=rloc),
            pltpu.VMEM((HIDDEN, 1000), jnp.float32),
            pltpu.VMEM((B_PER_W + LANES,), jnp.int32),
        )
        pl.run_scoped(
            functools.partial(score_phase, rloc),
            pltpu.VMEM((B_PER_W * HIDDEN,), jnp.float32),
            pltpu.VMEM((B_PER_W * HIDDEN,), jnp.float32),
            pltpu.VMEM((LANES * PSTRIDE,), jnp.float32),
            pltpu.VMEM((B_PER_W,), jnp.float32),
        )

    pl.run_scoped(late_phases,
                  pltpu.VMEM((B_PER_W * HIDDEN,), jnp.float32))


_mesh = plsc.VectorSubcoreMesh(
    core_axis_name="c", subcore_axis_name="s",
    num_cores=NUM_CORES, num_subcores=NUM_SUBCORES)

_params = pltpu.CompilerParams(
    needs_layout_passes=False, use_tc_tiling_on_sc=True)

_scan_call = functools.partial(
    pl.kernel,
    out_type=jax.ShapeDtypeStruct((2 * BATCH * HIDDEN,), jnp.float32),
    mesh=_mesh,
    compiler_params=_params,
    scratch_types=[
        pltpu.VMEM((NSLOT * HIDDEN,), jnp.float32),
        pltpu.SemaphoreType.DMA,
        pltpu.SemaphoreType.DMA,
    ],
)(_scan_body)

_score_call = functools.partial(
    pl.kernel,
    out_type=jax.ShapeDtypeStruct((BATCH,), jnp.float32),
    mesh=_mesh,
    compiler_params=_params,
    scratch_types=[
        pltpu.VMEM((NSLOT * HIDDEN,), jnp.float32),
        pltpu.SemaphoreType.DMA,
        pltpu.SemaphoreType.DMA,
    ],
)(_score_body)


@jax.jit
def kernel(head_index, rel_type, tail_index, node_emb, rel_emb):
    h = head_index.astype(jnp.int32)
    r = rel_type.astype(jnp.int32)
    t = tail_index.astype(jnp.int32)
    node_t = node_emb.T
    tail_t = node_emb.T[:, TAIL_START:]
    rel_t = rel_emb.T
    rows = _scan_call(node_t, tail_t, h, t)
    return _score_call(rows, rel_t, r)
